# baseline (speedup 1.0000x reference)
import jax
import jax.numpy as jnp
from jax import lax
from jax.experimental import pallas as pl
from jax.experimental.pallas import tpu as pltpu
from jax.experimental.pallas import tpu_sc as plsc

_T, _N, _E, _D = 4, 10000, 320000, 128
_NC, _NS, _L = 2, 16, 16
_EPW = _E // _NS
_K = 32
_NBG = 6
_CPB = 48
_IB = _CPB * _K
_NIB = _EPW // _IB
_TAIL = _EPW - _NIB * _IB
_RPW = 632
_RPW_LAST = _N - _RPW * (_NS - 1)


def _bcast_lane(vec16, l):
    idx = jnp.full((_L, 1), l, jnp.int32)
    dn = lax.GatherDimensionNumbers(offset_dims=(), collapsed_slice_dims=(0,),
                                    start_index_map=(0,))
    return lax.gather(vec16, idx, dn, (1,),
                      mode=lax.GatherScatterMode.PROMISE_IN_BOUNDS)


def _mm_body(x_ref, w_ref, o_ref):
    o_ref[...] = jnp.dot(x_ref[...], w_ref[...],
                         preferred_element_type=jnp.float32)


def _xw_matmul(x_flat, W):
    BN = 2000
    return pl.pallas_call(
        _mm_body,
        grid=(x_flat.shape[0] // BN,),
        in_specs=[
            pl.BlockSpec((BN, _D), lambda i: (i, 0)),
            pl.BlockSpec((_D, _D), lambda i: (0, 0)),
        ],
        out_specs=pl.BlockSpec((BN, _D), lambda i: (i, 0)),
        out_shape=jax.ShapeDtypeStruct((x_flat.shape[0], _D), jnp.float32),
    )(x_flat, W)


def _spmm_body(dst_hbm, src_hbm, val_hbm, xw_hbm, out_hbm,
               acc, src_b, dst_b, val_b, dst_v0, dst_v1, dst_tail_v,
               g0, g1, g2, g3, g4, g5, stg0, stg1,
               sg0, sg1, sg2, sg3, sg4, sg5, ss0, ss1, sem_i):
    c = lax.axis_index("c")
    s = lax.axis_index("s")
    gbufs = [g0, g1, g2, g3, g4, g5]
    gsems = [sg0, sg1, sg2, sg3, sg4, sg5]
    stgs = [stg0, stg1]
    ssems = [ss0, ss1]
    dvs = [dst_v0, dst_v1]

    def _copy_dst(off_e, dvr):
        for j in range(_K // _L):
            dvr[pl.ds(j * _L, _L)] = dst_b[pl.ds(off_e + j * _L, _L)]

    def _scale(rows, stg, off_e, ngroups=_K // _L):
        def _sc(gg, c2):
            val16 = val_b[pl.ds(off_e + gg * _L, _L)]
            for l in range(_L):
                bc = _bcast_lane(val16, l)
                k = gg * _L + l
                for j in range(_D // _L):
                    stg[k, pl.ds(j * _L, _L)] = (
                        rows[k, pl.ds(j * _L, _L)] * bc)
            return c2
        lax.fori_loop(0, ngroups, _sc, 0)

    def _issue_gather(off_e, rows, sem):
        pltpu.async_copy(xw_hbm.at[src_b.at[pl.ds(off_e, _K)]], rows, sem)

    def _wait_gather(rows, sem):
        pltpu.make_async_copy(xw_hbm.at[pl.ds(0, _K)], rows, sem).wait()

    def _issue_scatter(stg, dvr, sem):
        pltpu.async_copy(stg, acc.at[dvr], sem, add=True)

    def _wait_scatter(stg, dvr, sem):
        pltpu.make_async_copy(stg, acc.at[dvr], sem).wait()

    def _load_idx_block(eb, n, t):
        d1 = pltpu.async_copy(src_hbm.at[pl.ds(eb, n)],
                              src_b.at[pl.ds(0, n)], sem_i)
        d2 = pltpu.async_copy(dst_hbm.at[pl.ds(eb, n)],
                              dst_b.at[pl.ds(0, n)], sem_i)
        d3 = pltpu.async_copy(val_hbm.at[pl.ds(eb, n)],
                              val_b.at[pl.ds(0, n)], sem_i)
        d1.wait(); d2.wait(); d3.wait()

        def _gl(i, carry):
            b = i * _L
            src_b[pl.ds(b, _L)] = src_b[pl.ds(b, _L)] + t * _N
            return carry
        lax.fori_loop(0, n // _L, _gl, 0)

    def _run_block(nch, first):
        ngrp = nch // _NBG
        for b in range(_NBG):
            _issue_gather(b * _K, gbufs[b], gsems[b])

        def _grp(g, carry):
            qb0 = g * _NBG
            for b in range(_NBG):
                sb = b % 2
                qb = qb0 + b
                _wait_gather(gbufs[b], gsems[b])
                if b < 2:
                    @pl.when(jnp.logical_not(
                        jnp.logical_and(first, g == 0)))
                    def _():
                        _wait_scatter(stgs[sb], dvs[sb], ssems[sb])
                else:
                    _wait_scatter(stgs[sb], dvs[sb], ssems[sb])
                _scale(gbufs[b], stgs[sb], qb * _K)

                @pl.when(g < ngrp - 1)
                def _():
                    _issue_gather((qb + _NBG) * _K, gbufs[b], gsems[b])
                _copy_dst(qb * _K, dvs[sb])
                _issue_scatter(stgs[sb], dvs[sb], ssems[sb])
            return carry
        lax.fori_loop(0, ngrp, _grp, 0)

    for tt in range(_T // _NC):
        t = c * (_T // _NC) + tt

        def _zf(k, carry):
            for j in range(_D // _L):
                stg0[k, pl.ds(j * _L, _L)] = jnp.zeros((_L,), jnp.float32)
            return carry
        lax.fori_loop(0, _K, _zf, 0)
        r0 = s * _RPW

        def _zero_stripe(rows):
            for q in range(rows // _K):
                pltpu.sync_copy(stg0, acc.at[pl.ds(r0 + q * _K, _K)])
            rem = rows - (rows // _K) * _K
            if rem:
                pltpu.sync_copy(stg0.at[pl.ds(0, rem)],
                                acc.at[pl.ds(r0 + (rows // _K) * _K, rem)])

        @pl.when(s < _NS - 1)
        def _():
            _zero_stripe(_RPW)

        @pl.when(s == _NS - 1)
        def _():
            _zero_stripe(_RPW_LAST)

        plsc.subcore_barrier()

        ebase = t * _E + s * _EPW

        def _block(ib, carry):
            _load_idx_block(ebase + ib * _IB, _IB, t)
            _run_block(_CPB, ib == 0)
            return carry
        lax.fori_loop(0, _NIB, _block, 0)

        _wait_scatter(stg0, dst_v0, ss0)
        _wait_scatter(stg1, dst_v1, ss1)

        _load_idx_block(ebase + _NIB * _IB, _TAIL, t)
        off_e = 0
        for j in range(_TAIL // _L):
            dst_tail_v[pl.ds(j * _L, _L)] = dst_b[pl.ds(off_e + j * _L, _L)]
        pltpu.async_copy(xw_hbm.at[src_b.at[pl.ds(off_e, _TAIL)]],
                         g0.at[pl.ds(0, _TAIL)], sg0).wait()
        _scale(g0, stg0, off_e, ngroups=_TAIL // _L)
        pltpu.sync_copy(stg0.at[pl.ds(0, _TAIL)], acc.at[dst_tail_v],
                        add=True)

        plsc.subcore_barrier()

        obase = t * _N + r0

        @pl.when(s < _NS - 1)
        def _():
            pltpu.sync_copy(acc.at[pl.ds(r0, _RPW)],
                            out_hbm.at[pl.ds(obase, _RPW)])

        @pl.when(s == _NS - 1)
        def _():
            pltpu.sync_copy(acc.at[pl.ds(r0, _RPW_LAST)],
                            out_hbm.at[pl.ds(obase, _RPW_LAST)])


def kernel(adj_indices, adj_values, input, M, W):
    dst = adj_indices[:, 0, :].reshape(-1)
    src = adj_indices[:, 1, :].reshape(-1)
    val = adj_values.reshape(-1)
    x_flat = input.reshape(_T * _N, _D)
    xw = _xw_matmul(x_flat, W)

    mesh = plsc.VectorSubcoreMesh(core_axis_name="c", subcore_axis_name="s")
    spmm = pl.kernel(
        _spmm_body,
        out_type=jax.ShapeDtypeStruct((_T * _N, _D), jnp.float32),
        compiler_params=pltpu.CompilerParams(use_tc_tiling_on_sc=False),
        mesh=mesh,
        scratch_types=[
            pltpu.VMEM_SHARED((_N, _D), jnp.float32),
            pltpu.VMEM((_IB,), jnp.int32),
            pltpu.VMEM((_IB,), jnp.int32),
            pltpu.VMEM((_IB,), jnp.float32),
            pltpu.VMEM((_K,), jnp.int32),
            pltpu.VMEM((_K,), jnp.int32),
            pltpu.VMEM((_TAIL,), jnp.int32),
            pltpu.VMEM((_K, _D), jnp.float32),
            pltpu.VMEM((_K, _D), jnp.float32),
            pltpu.VMEM((_K, _D), jnp.float32),
            pltpu.VMEM((_K, _D), jnp.float32),
            pltpu.VMEM((_K, _D), jnp.float32),
            pltpu.VMEM((_K, _D), jnp.float32),
            pltpu.VMEM((_K, _D), jnp.float32),
            pltpu.VMEM((_K, _D), jnp.float32),
            pltpu.SemaphoreType.DMA,
            pltpu.SemaphoreType.DMA,
            pltpu.SemaphoreType.DMA,
            pltpu.SemaphoreType.DMA,
            pltpu.SemaphoreType.DMA,
            pltpu.SemaphoreType.DMA,
            pltpu.SemaphoreType.DMA,
            pltpu.SemaphoreType.DMA,
            pltpu.SemaphoreType.DMA,
        ],
    )
    out = spmm(dst, src, val, xw)
    return out.reshape(_T, _N, _D)


# speedup vs baseline: 1.0048x; 1.0048x over previous
import jax
import jax.numpy as jnp
from jax import lax
from jax.experimental import pallas as pl
from jax.experimental.pallas import tpu as pltpu
from jax.experimental.pallas import tpu_sc as plsc

_T, _N, _E, _D = 4, 10000, 320000, 128
_NC, _NS, _L = 2, 16, 16
_EPW = _E // _NS
_K = 32
_NBG = 4
_CPB = 48
_IB = _CPB * _K
_NIB = _EPW // _IB
_TAIL = _EPW - _NIB * _IB
_RPW = 632
_RPW_LAST = _N - _RPW * (_NS - 1)


def _bcast_lane(vec16, l):
    idx = jnp.full((_L, 1), l, jnp.int32)
    dn = lax.GatherDimensionNumbers(offset_dims=(), collapsed_slice_dims=(0,),
                                    start_index_map=(0,))
    return lax.gather(vec16, idx, dn, (1,),
                      mode=lax.GatherScatterMode.PROMISE_IN_BOUNDS)


def _mm_body(x_ref, w_ref, o_ref):
    o_ref[...] = jnp.dot(x_ref[...], w_ref[...],
                         preferred_element_type=jnp.float32)


def _xw_matmul(x_flat, W):
    BN = 2000
    return pl.pallas_call(
        _mm_body,
        grid=(x_flat.shape[0] // BN,),
        in_specs=[
            pl.BlockSpec((BN, _D), lambda i: (i, 0)),
            pl.BlockSpec((_D, _D), lambda i: (0, 0)),
        ],
        out_specs=pl.BlockSpec((BN, _D), lambda i: (i, 0)),
        out_shape=jax.ShapeDtypeStruct((x_flat.shape[0], _D), jnp.float32),
    )(x_flat, W)


def _spmm_body(dst_hbm, src_hbm, val_hbm, xw_hbm, out_hbm,
               acc, src_b, dst_b, val_b, dst_v0, dst_v1, dst_tail_v,
               g0, g1, g2, g3, stg0, stg1,
               sg0, sg1, sg2, sg3, ss0, ss1, sem_i):
    c = lax.axis_index("c")
    s = lax.axis_index("s")
    gbufs = [g0, g1, g2, g3]
    gsems = [sg0, sg1, sg2, sg3]
    stgs = [stg0, stg1]
    ssems = [ss0, ss1]
    dvs = [dst_v0, dst_v1]

    def _copy_dst(off_e, dvr):
        for j in range(_K // _L):
            dvr[pl.ds(j * _L, _L)] = dst_b[pl.ds(off_e + j * _L, _L)]

    def _scale(rows, stg, off_e, ngroups=_K // _L):
        def _sc(gg, c2):
            val16 = val_b[pl.ds(off_e + gg * _L, _L)]
            for l in range(_L):
                bc = _bcast_lane(val16, l)
                k = gg * _L + l
                for j in range(_D // _L):
                    stg[k, pl.ds(j * _L, _L)] = (
                        rows[k, pl.ds(j * _L, _L)] * bc)
            return c2
        lax.fori_loop(0, ngroups, _sc, 0)

    def _issue_gather(off_e, rows, sem):
        pltpu.async_copy(xw_hbm.at[src_b.at[pl.ds(off_e, _K)]], rows, sem)

    def _wait_gather(rows, sem):
        pltpu.make_async_copy(xw_hbm.at[pl.ds(0, _K)], rows, sem).wait()

    def _issue_scatter(stg, dvr, sem):
        pltpu.async_copy(stg, acc.at[dvr], sem, add=True)

    def _wait_scatter(stg, dvr, sem):
        pltpu.make_async_copy(stg, acc.at[dvr], sem).wait()

    def _load_idx_block(eb, n, t):
        d1 = pltpu.async_copy(src_hbm.at[pl.ds(eb, n)],
                              src_b.at[pl.ds(0, n)], sem_i)
        d2 = pltpu.async_copy(dst_hbm.at[pl.ds(eb, n)],
                              dst_b.at[pl.ds(0, n)], sem_i)
        d3 = pltpu.async_copy(val_hbm.at[pl.ds(eb, n)],
                              val_b.at[pl.ds(0, n)], sem_i)
        d1.wait(); d2.wait(); d3.wait()

        def _gl(i, carry):
            b = i * _L
            src_b[pl.ds(b, _L)] = src_b[pl.ds(b, _L)] + t * _N
            return carry
        lax.fori_loop(0, n // _L, _gl, 0)

    def _run_block(nch, first):
        ngrp = nch // _NBG
        for b in range(_NBG):
            _issue_gather(b * _K, gbufs[b], gsems[b])

        def _grp(g, carry):
            qb0 = g * _NBG
            for b in range(_NBG):
                sb = b % 2
                qb = qb0 + b
                _wait_gather(gbufs[b], gsems[b])
                if b < 2:
                    @pl.when(jnp.logical_not(
                        jnp.logical_and(first, g == 0)))
                    def _():
                        _wait_scatter(stgs[sb], dvs[sb], ssems[sb])
                else:
                    _wait_scatter(stgs[sb], dvs[sb], ssems[sb])
                _scale(gbufs[b], stgs[sb], qb * _K)

                @pl.when(g < ngrp - 1)
                def _():
                    _issue_gather((qb + _NBG) * _K, gbufs[b], gsems[b])
                _copy_dst(qb * _K, dvs[sb])
                _issue_scatter(stgs[sb], dvs[sb], ssems[sb])
            return carry
        lax.fori_loop(0, ngrp, _grp, 0)

    for tt in range(_T // _NC):
        t = c * (_T // _NC) + tt

        def _zf(k, carry):
            for j in range(_D // _L):
                stg0[k, pl.ds(j * _L, _L)] = jnp.zeros((_L,), jnp.float32)
            return carry
        lax.fori_loop(0, _K, _zf, 0)
        r0 = s * _RPW

        def _zero_stripe(rows):
            for q in range(rows // _K):
                pltpu.sync_copy(stg0, acc.at[pl.ds(r0 + q * _K, _K)])
            rem = rows - (rows // _K) * _K
            if rem:
                pltpu.sync_copy(stg0.at[pl.ds(0, rem)],
                                acc.at[pl.ds(r0 + (rows // _K) * _K, rem)])

        @pl.when(s < _NS - 1)
        def _():
            _zero_stripe(_RPW)

        @pl.when(s == _NS - 1)
        def _():
            _zero_stripe(_RPW_LAST)

        plsc.subcore_barrier()

        ebase = t * _E + s * _EPW

        def _block(ib, carry):
            _load_idx_block(ebase + ib * _IB, _IB, t)
            _run_block(_CPB, ib == 0)
            return carry
        lax.fori_loop(0, _NIB, _block, 0)

        _wait_scatter(stg0, dst_v0, ss0)
        _wait_scatter(stg1, dst_v1, ss1)

        _load_idx_block(ebase + _NIB * _IB, _TAIL, t)
        off_e = 0
        for j in range(_TAIL // _L):
            dst_tail_v[pl.ds(j * _L, _L)] = dst_b[pl.ds(off_e + j * _L, _L)]
        pltpu.async_copy(xw_hbm.at[src_b.at[pl.ds(off_e, _TAIL)]],
                         g0.at[pl.ds(0, _TAIL)], sg0).wait()
        _scale(g0, stg0, off_e, ngroups=_TAIL // _L)
        pltpu.sync_copy(stg0.at[pl.ds(0, _TAIL)], acc.at[dst_tail_v],
                        add=True)

        plsc.subcore_barrier()

        obase = t * _N + r0

        @pl.when(s < _NS - 1)
        def _():
            pltpu.sync_copy(acc.at[pl.ds(r0, _RPW)],
                            out_hbm.at[pl.ds(obase, _RPW)])

        @pl.when(s == _NS - 1)
        def _():
            pltpu.sync_copy(acc.at[pl.ds(r0, _RPW_LAST)],
                            out_hbm.at[pl.ds(obase, _RPW_LAST)])


def kernel(adj_indices, adj_values, input, M, W):
    dst = adj_indices[:, 0, :].reshape(-1)
    src = adj_indices[:, 1, :].reshape(-1)
    val = adj_values.reshape(-1)
    x_flat = input.reshape(_T * _N, _D)
    xw = _xw_matmul(x_flat, W)

    mesh = plsc.VectorSubcoreMesh(core_axis_name="c", subcore_axis_name="s")
    spmm = pl.kernel(
        _spmm_body,
        out_type=jax.ShapeDtypeStruct((_T * _N, _D), jnp.float32),
        compiler_params=pltpu.CompilerParams(use_tc_tiling_on_sc=False),
        mesh=mesh,
        scratch_types=[
            pltpu.VMEM_SHARED((_N, _D), jnp.float32),
            pltpu.VMEM((_IB,), jnp.int32),
            pltpu.VMEM((_IB,), jnp.int32),
            pltpu.VMEM((_IB,), jnp.float32),
            pltpu.VMEM((_K,), jnp.int32),
            pltpu.VMEM((_K,), jnp.int32),
            pltpu.VMEM((_TAIL,), jnp.int32),
            pltpu.VMEM((_K, _D), jnp.float32),
            pltpu.VMEM((_K, _D), jnp.float32),
            pltpu.VMEM((_K, _D), jnp.float32),
            pltpu.VMEM((_K, _D), jnp.float32),
            pltpu.VMEM((_K, _D), jnp.float32),
            pltpu.VMEM((_K, _D), jnp.float32),
            pltpu.SemaphoreType.DMA,
            pltpu.SemaphoreType.DMA,
            pltpu.SemaphoreType.DMA,
            pltpu.SemaphoreType.DMA,
            pltpu.SemaphoreType.DMA,
            pltpu.SemaphoreType.DMA,
            pltpu.SemaphoreType.DMA,
        ],
    )
    out = spmm(dst, src, val, xw)
    return out.reshape(_T, _N, _D)
